# baseline (device time: 43434 ns/iter reference)
import jax
import jax.numpy as jnp
from jax import lax
from jax.experimental import pallas as pl
from jax.experimental.pallas import tpu as pltpu

N_DEV = 8
N_LAYERS = 3


def kernel(x, Win0, Wout0, Win1, Wout1, Win2, Wout2):
    b, d_shard = x.shape
    h_dim = Win0.shape[1]

    def body(x_ref, win0_ref, wout0_ref, win1_ref, wout1_ref, win2_ref,
             wout2_ref, out_ref, comm_ref, send_sems, recv_sems):
        my_pos = lax.axis_index("i")

        barrier_sem = pltpu.get_barrier_semaphore()
        for d in range(1, N_DEV):
            pl.semaphore_signal(
                barrier_sem, inc=1,
                device_id=((my_pos + d) % N_DEV,),
                device_id_type=pl.DeviceIdType.MESH,
            )
        pl.semaphore_wait(barrier_sem, N_DEV - 1)

        wins = [win0_ref, win1_ref, win2_ref]
        wouts = [wout0_ref, wout1_ref, wout2_ref]

        x_cur = x_ref[:, :].astype(jnp.bfloat16)
        for l in range(N_LAYERS):
            w_in = wins[l][:, :].astype(jnp.bfloat16)
            partial = jnp.dot(x_cur, w_in, preferred_element_type=jnp.float32)
            comm_ref[l, 0, :, :] = partial.astype(jnp.bfloat16)

            rdmas = []
            for d in range(1, N_DEV):
                rdma = pltpu.make_async_remote_copy(
                    src_ref=comm_ref.at[l, 0],
                    dst_ref=comm_ref.at[l, d],
                    send_sem=send_sems.at[l, d - 1],
                    recv_sem=recv_sems.at[l, d - 1],
                    device_id=((my_pos + d) % N_DEV,),
                    device_id_type=pl.DeviceIdType.MESH,
                )
                rdma.start()
                rdmas.append(rdma)

            for rdma in rdmas:
                rdma.wait_recv()

            h = comm_ref[l, 0, :, :].astype(jnp.float32)
            for d in range(1, N_DEV):
                h = h + comm_ref[l, d, :, :].astype(jnp.float32)
            h = jnp.maximum(h, 0.0).astype(jnp.bfloat16)

            w_out = wouts[l][:, :].astype(jnp.bfloat16)
            nxt = jnp.dot(h, w_out, preferred_element_type=jnp.float32)
            if l == N_LAYERS - 1:
                out_ref[:, :] = nxt
            else:
                x_cur = nxt.astype(jnp.bfloat16)

            for rdma in rdmas:
                rdma.wait_send()

    return pl.pallas_call(
        body,
        out_shape=jax.ShapeDtypeStruct((b, d_shard), jnp.float32),
        in_specs=[pl.BlockSpec(memory_space=pltpu.VMEM)] * 7,
        out_specs=pl.BlockSpec(memory_space=pltpu.VMEM),
        scratch_shapes=[
            pltpu.VMEM((N_LAYERS, N_DEV, b, h_dim), jnp.bfloat16),
            pltpu.SemaphoreType.DMA((N_LAYERS, N_DEV - 1)),
            pltpu.SemaphoreType.DMA((N_LAYERS, N_DEV - 1)),
        ],
        compiler_params=pltpu.CompilerParams(collective_id=0),
    )(x, Win0, Wout0, Win1, Wout1, Win2, Wout2)


# device time: 37482 ns/iter; 1.1588x vs baseline; 1.1588x over previous
import jax
import jax.numpy as jnp
from jax import lax
from jax.experimental import pallas as pl
from jax.experimental.pallas import tpu as pltpu

N_DEV = 8
N_LAYERS = 3


def kernel(x, Win0, Wout0, Win1, Wout1, Win2, Wout2):
    b, d_shard = x.shape
    h_dim = Win0.shape[1]
    blk = h_dim // N_DEV

    def body(x_ref, win0_ref, wout0_ref, win1_ref, wout1_ref, win2_ref,
             wout2_ref, out_ref, part_ref, rs_ref, ag_ref,
             send_sems, recv_sems):
        my_pos = lax.axis_index("i")

        barrier_sem = pltpu.get_barrier_semaphore()
        for d in range(1, N_DEV):
            pl.semaphore_signal(
                barrier_sem, inc=1,
                device_id=((my_pos + d) % N_DEV,),
                device_id_type=pl.DeviceIdType.MESH,
            )
        pl.semaphore_wait(barrier_sem, N_DEV - 1)

        wins = [win0_ref, win1_ref, win2_ref]
        wouts = [wout0_ref, wout1_ref, wout2_ref]

        x_cur = x_ref[:, :].astype(jnp.bfloat16)
        for l in range(N_LAYERS):
            w_in = wins[l][:, :].astype(jnp.bfloat16)
            partial = jnp.dot(x_cur, w_in, preferred_element_type=jnp.float32)
            part_ref[l, :, :] = partial.astype(jnp.bfloat16)

            rs_rdmas = []
            for d in range(1, N_DEV):
                t = (my_pos + d) % N_DEV
                rdma = pltpu.make_async_remote_copy(
                    src_ref=part_ref.at[l, :, pl.ds(t * blk, blk)],
                    dst_ref=rs_ref.at[l, d - 1],
                    send_sem=send_sems.at[l, 0, d - 1],
                    recv_sem=recv_sems.at[l, 0, d - 1],
                    device_id=(t,),
                    device_id_type=pl.DeviceIdType.MESH,
                )
                rdma.start()
                rs_rdmas.append(rdma)

            acc = part_ref[l, :, pl.ds(my_pos * blk, blk)].astype(jnp.float32)
            for d in range(1, N_DEV):
                rs_rdmas[d - 1].wait_recv()
                acc = acc + rs_ref[l, d - 1, :, :].astype(jnp.float32)
            hred = jnp.maximum(acc, 0.0).astype(jnp.bfloat16)
            ag_ref[l, :, pl.ds(my_pos * blk, blk)] = hred

            ag_rdmas = []
            for d in range(1, N_DEV):
                t = (my_pos + d) % N_DEV
                rdma = pltpu.make_async_remote_copy(
                    src_ref=ag_ref.at[l, :, pl.ds(my_pos * blk, blk)],
                    dst_ref=ag_ref.at[l, :, pl.ds(my_pos * blk, blk)],
                    send_sem=send_sems.at[l, 1, d - 1],
                    recv_sem=recv_sems.at[l, 1, d - 1],
                    device_id=(t,),
                    device_id_type=pl.DeviceIdType.MESH,
                )
                rdma.start()
                ag_rdmas.append(rdma)
            for rdma in ag_rdmas:
                rdma.wait_recv()

            h = ag_ref[l, :, :]
            w_out = wouts[l][:, :].astype(jnp.bfloat16)
            nxt = jnp.dot(h, w_out, preferred_element_type=jnp.float32)
            if l == N_LAYERS - 1:
                out_ref[:, :] = nxt
            else:
                x_cur = nxt.astype(jnp.bfloat16)

            for rdma in rs_rdmas:
                rdma.wait_send()
            for rdma in ag_rdmas:
                rdma.wait_send()

    return pl.pallas_call(
        body,
        out_shape=jax.ShapeDtypeStruct((b, d_shard), jnp.float32),
        in_specs=[pl.BlockSpec(memory_space=pltpu.VMEM)] * 7,
        out_specs=pl.BlockSpec(memory_space=pltpu.VMEM),
        scratch_shapes=[
            pltpu.VMEM((N_LAYERS, b, h_dim), jnp.bfloat16),
            pltpu.VMEM((N_LAYERS, N_DEV - 1, b, blk), jnp.bfloat16),
            pltpu.VMEM((N_LAYERS, b, h_dim), jnp.bfloat16),
            pltpu.SemaphoreType.DMA((N_LAYERS, 2, N_DEV - 1)),
            pltpu.SemaphoreType.DMA((N_LAYERS, 2, N_DEV - 1)),
        ],
        compiler_params=pltpu.CompilerParams(collective_id=0),
    )(x, Win0, Wout0, Win1, Wout1, Win2, Wout2)
